# baseline (speedup 1.0000x reference)
import jax
import jax.numpy as jnp
from jax.experimental import pallas as pl
from jax.experimental.pallas import tpu as pltpu

N = 10000
DIN = 128
DOUT = 128
BM = 400
NUM_BLOCKS = N // BM


def _gcn_kernel(
    x_ref, adj_ref, w_ref, ws_ref, b_ref, gamma_ref, beta_ref,
    out_ref, sup_ref, sum_ref, sq_ref,
):
    i = pl.program_id(0)

    @pl.when(i == 0)
    def _init():
        sup_ref[...] = jnp.dot(
            x_ref[...], w_ref[...], preferred_element_type=jnp.float32
        )
        sum_ref[...] = jnp.zeros_like(sum_ref)
        sq_ref[...] = jnp.zeros_like(sq_ref)

    rows = pl.ds(i * BM, BM)
    o = (
        jnp.dot(adj_ref[...], sup_ref[...], preferred_element_type=jnp.float32)
        + jnp.dot(x_ref[rows, :], ws_ref[...], preferred_element_type=jnp.float32)
        + b_ref[...]
    )
    out_ref[rows, :] = o
    sum_ref[...] += jnp.sum(o, axis=0, keepdims=True)
    sq_ref[...] += jnp.sum(o * o, axis=0, keepdims=True)

    @pl.when(i == NUM_BLOCKS - 1)
    def _normalize():
        mean = sum_ref[...] * (1.0 / N)
        var = sq_ref[...] * (1.0 / N) - mean * mean
        scale = gamma_ref[...] * jax.lax.rsqrt(var + 1e-5)
        shift = beta_ref[...] - mean * scale
        out_ref[...] = out_ref[...] * scale + shift


@jax.jit
def kernel(x, adj, W, W_self, b, gamma, beta):
    b2 = b.reshape(1, DOUT)
    gamma2 = gamma.reshape(1, DOUT)
    beta2 = beta.reshape(1, DOUT)

    out = pl.pallas_call(
        _gcn_kernel,
        grid=(NUM_BLOCKS,),
        in_specs=[
            pl.BlockSpec((N, DIN), lambda i: (0, 0)),
            pl.BlockSpec((BM, N), lambda i: (i, 0)),
            pl.BlockSpec((DIN, DOUT), lambda i: (0, 0)),
            pl.BlockSpec((DIN, DOUT), lambda i: (0, 0)),
            pl.BlockSpec((1, DOUT), lambda i: (0, 0)),
            pl.BlockSpec((1, DOUT), lambda i: (0, 0)),
            pl.BlockSpec((1, DOUT), lambda i: (0, 0)),
        ],
        out_specs=pl.BlockSpec((N, DOUT), lambda i: (0, 0)),
        out_shape=jax.ShapeDtypeStruct((N, DOUT), jnp.float32),
        scratch_shapes=[
            pltpu.VMEM((N, DIN), jnp.float32),
            pltpu.VMEM((1, DOUT), jnp.float32),
            pltpu.VMEM((1, DOUT), jnp.float32),
        ],
    )(x, adj, W, W_self, b2, gamma2, beta2)

    return out
